# Initial kernel scaffold; baseline (speedup 1.0000x reference)
import jax
import jax.numpy as jnp
from jax.experimental import pallas as pl


def kernel(arg0_1):
    raise NotImplementedError("write your pallas kernel here")


# speedup vs baseline: 1.5946x; 1.5946x over previous
import functools

import numpy as np
import jax
import jax.numpy as jnp
from jax import lax
from jax.experimental import pallas as pl
from jax.experimental.pallas import tpu as pltpu, tpu_sc as plsc

IH, IW, OH, OW = 345, 456, 271, 272
BC = 6
TASKS = BC * OH
NSUB = 32
TPW = 51
NR = 72
FLATROWS = BC * IH
NTW = 4
NTH = 3
NG = OW // 16


def _resize_weights(in_size, out_size):
    scale = in_size / out_size
    inv = out_size / in_size
    i = np.arange(out_size, dtype=np.float32)
    center = (i + np.float32(0.5)) * np.float32(scale)
    xmin = np.maximum((center - np.float32(scale) + np.float32(0.5)).astype(np.int32), 0)
    xmax = np.minimum((center + np.float32(scale) + np.float32(0.5)).astype(np.int32), in_size)
    ksize = np.minimum(xmax - xmin, 5)
    j = np.arange(5, dtype=np.int32)[:, None]
    dist = (j.astype(np.float32) + xmin[None].astype(np.float32) - center[None]
            + np.float32(0.5)) * np.float32(inv)
    w = np.float32(1.0) - np.minimum(np.abs(dist), np.float32(1.0))
    w = np.where(j < ksize[None], w, np.float32(0.0)).astype(np.float32)
    w = (w / w.sum(0, dtype=np.float32)).astype(np.float32)
    w[4] = 0.0
    idx = np.minimum(xmin[None] + j, in_size - 1).astype(np.int32)
    return idx, w


def _build_tables():
    idxW, wW = _resize_weights(IW, OW)
    idxH, wH = _resize_weights(IH, OH)
    assert np.all(wW[NTW:] == 0) and np.all(wH[NTH:] == 0)

    wci = np.zeros((NG, NTW, 16), np.int32)
    wcw = np.zeros((NG, NTW, 16), np.float32)
    for g in range(NG):
        cols = slice(g * 16, g * 16 + 16)
        for u in range(NTW):
            wci[g, u] = idxW[u, cols]
            wcw[g, u] = wW[u, cols]

    loc = np.zeros((NSUB, TPW, NTH, 16), np.int32)
    hw = np.zeros((NSUB, TPW, NTH, 16), np.float32)
    for w in range(NSUB):
        tau0 = min(w * TPW, TASKS - TPW)
        rstart = max(0, min((tau0 * IH) // OH - 2, FLATROWS - NR))
        for k in range(TPW):
            bc, h = divmod(tau0 + k, OH)
            for t in range(NTH):
                l = bc * IH + int(idxH[t, h]) - rstart
                if wH[t, h] == 0.0:
                    l = min(max(l, 0), NR - 1)
                assert 0 <= l < NR
                loc[w, k, t] = l * OW
                hw[w, k, t] = wH[t, h]
    return wci, wcw, loc, hw


_WCI, _WCW, _LOC, _HW = _build_tables()

_mesh = plsc.VectorSubcoreMesh(core_axis_name="c", subcore_axis_name="s")


@functools.partial(
    pl.kernel,
    out_type=jax.ShapeDtypeStruct((TASKS * OW,), jnp.float32),
    mesh=_mesh,
    compiler_params=pltpu.CompilerParams(
        needs_layout_passes=False, use_tc_tiling_on_sc=False),
    scratch_types=[
        pltpu.VMEM((NR * IW,), jnp.float32),
        pltpu.VMEM((NR * OW,), jnp.float32),
        pltpu.VMEM((TPW * OW,), jnp.float32),
        pltpu.VMEM((TPW, NTH, 16), jnp.int32),
        pltpu.VMEM((TPW, NTH, 16), jnp.float32),
        pltpu.VMEM((NG, NTW, 16), jnp.int32),
        pltpu.VMEM((NG, NTW, 16), jnp.float32),
    ],
)
def _resize_sc(x_hbm, loc_hbm, hw_hbm, wci_hbm, wcw_hbm, out_hbm,
               block_v, wrows_v, out_v, loc_v, hw_v, wci_v, wcw_v):
    nc = 2
    wid = lax.axis_index("s") * nc + lax.axis_index("c")
    tau0 = jnp.minimum(wid * TPW, TASKS - TPW)
    rstart = jnp.maximum(0, jnp.minimum((tau0 * IH) // OH - 2, FLATROWS - NR))

    pltpu.sync_copy(x_hbm.at[pl.ds(rstart * IW, NR * IW)], block_v)
    pltpu.sync_copy(loc_hbm.at[wid], loc_v)
    pltpu.sync_copy(hw_hbm.at[wid], hw_v)
    pltpu.sync_copy(wci_hbm, wci_v)
    pltpu.sync_copy(wcw_hbm, wcw_v)

    for g in range(NG):
        ci = [wci_v[g, u] for u in range(NTW)]
        cw = [wcw_v[g, u] for u in range(NTW)]

        def wbody(r, _, ci=ci, cw=cw, g=g):
            rb = r * IW
            acc = plsc.load_gather(block_v, [ci[0] + rb]) * cw[0]
            for u in range(1, NTW):
                acc = acc + plsc.load_gather(block_v, [ci[u] + rb]) * cw[u]
            wrows_v[pl.ds(r * OW + g * 16, 16)] = acc
            return 0

        lax.fori_loop(0, NR, wbody, 0)

    lane = lax.iota(jnp.int32, 16)

    def hbody(k, _):
        rows = [loc_v[k, t] for t in range(NTH)]
        wts = [hw_v[k, t] for t in range(NTH)]
        for g in range(NG):
            cvec = lane + g * 16
            acc = plsc.load_gather(wrows_v, [rows[0] + cvec]) * wts[0]
            for t in range(1, NTH):
                acc = acc + plsc.load_gather(wrows_v, [rows[t] + cvec]) * wts[t]
            out_v[pl.ds(k * OW + g * 16, 16)] = acc
        return 0

    lax.fori_loop(0, TPW, hbody, 0)

    pltpu.sync_copy(out_v, out_hbm.at[pl.ds(tau0 * OW, TPW * OW)])


def kernel(arg0_1):
    out = _resize_sc(arg0_1.reshape(-1), _LOC, _HW, _WCI, _WCW)
    return out.reshape(2, 3, OH, OW)
